# Initial kernel scaffold; baseline (speedup 1.0000x reference)
import jax
import jax.numpy as jnp
from jax.experimental import pallas as pl


def kernel(states, graph_ids, gate_W, gate_b, out_W, out_b):
    raise NotImplementedError("write your pallas kernel here")


# speedup vs baseline: 7.1307x; 7.1307x over previous
import functools

import jax
import jax.numpy as jnp
from jax import lax
from jax.experimental import pallas as pl
from jax.experimental.pallas import tpu as pltpu
from jax.experimental.pallas import tpu_sc as plsc

N = 100000
D = 512
G = 512
GPS = 520
BLK = 1024
NBLK = 100
NC, NS = 2, 16
NW = NC * NS
CS = 128
NCH = 25
CPT = NCH * CS
NPAD = NW * CPT
LAST_X_BLK = (N - 1) // BLK


def _stage1_body(x_ref, wt_ref, b_ref, p0_ref, p1_ref, p2_ref):
    x = x_ref[...]
    yt = lax.dot_general(wt_ref[...], x, (((1,), (1,)), ((), ())),
                         preferred_element_type=jnp.float32)
    yt = yt + b_ref[...]
    vals = jnp.exp(yt[0:1])
    p0_ref[0] = vals
    p1_ref[0] = yt[1:2] * vals
    p2_ref[0] = yt[2:3] * vals


def _stage2_body(v0_hbm, v1_hbm, v2_hbm, ids_hbm, zeros_hbm, part_hbm,
                 v0_v, v1_v, v2_v, ids_v, stage_v, acc0, acc1, acc2):
    c = lax.axis_index("c")
    s = lax.axis_index("s")
    w = s * NC + c
    base = w * CPT

    @pl.when(s == 0)
    def _zero():
        pltpu.sync_copy(zeros_hbm, stage_v)
        pltpu.sync_copy(stage_v, acc0)
        pltpu.sync_copy(stage_v, acc1)
        pltpu.sync_copy(stage_v, acc2)

    plsc.subcore_barrier()
    pltpu.sync_copy(v0_hbm.at[pl.ds(base, CPT)], v0_v)
    pltpu.sync_copy(v1_hbm.at[pl.ds(base, CPT)], v1_v)
    pltpu.sync_copy(v2_hbm.at[pl.ds(base, CPT)], v2_v)
    pltpu.sync_copy(ids_hbm.at[w], ids_v)
    for j in range(NCH):
        idsj = ids_v.at[j]
        sl = pl.ds(j * CS, CS)
        pltpu.sync_copy(v0_v.at[sl], acc0.at[idsj], add=True)
        pltpu.sync_copy(v1_v.at[sl], acc1.at[idsj], add=True)
        pltpu.sync_copy(v2_v.at[sl], acc2.at[idsj], add=True)
    plsc.subcore_barrier()

    @pl.when(s == 0)
    def _emit():
        pltpu.sync_copy(acc0, stage_v)
        pltpu.sync_copy(stage_v, part_hbm.at[pl.ds(c * 3 * GPS, GPS)])
        pltpu.sync_copy(acc1, stage_v)
        pltpu.sync_copy(stage_v, part_hbm.at[pl.ds((c * 3 + 1) * GPS, GPS)])
        pltpu.sync_copy(acc2, stage_v)
        pltpu.sync_copy(stage_v, part_hbm.at[pl.ds((c * 3 + 2) * GPS, GPS)])


_stage2 = functools.partial(
    pl.kernel,
    out_type=jax.ShapeDtypeStruct((NC * 3 * GPS,), jnp.float32),
    mesh=plsc.VectorSubcoreMesh(core_axis_name="c", subcore_axis_name="s"),
    scratch_types=[
        pltpu.VMEM((CPT,), jnp.float32),
        pltpu.VMEM((CPT,), jnp.float32),
        pltpu.VMEM((CPT,), jnp.float32),
        pltpu.VMEM((NCH, CS), jnp.int32),
        pltpu.VMEM((GPS,), jnp.float32),
        pltpu.VMEM_SHARED((GPS,), jnp.float32),
        pltpu.VMEM_SHARED((GPS,), jnp.float32),
        pltpu.VMEM_SHARED((GPS,), jnp.float32),
    ],
)(_stage2_body)


def _stage3_body(part_ref, out_ref):
    a = part_ref[0] + part_ref[1]
    gr = a[1:3, 0:G] / (a[0:1, 0:G] + 1e-16)
    e2 = (lax.broadcasted_iota(jnp.int32, (2, 2), 0)
          == lax.broadcasted_iota(jnp.int32, (2, 2), 1)).astype(jnp.float32)
    out_ref[...] = lax.dot_general(gr, e2, (((0,), (0,)), ((), ())),
                                   preferred_element_type=jnp.float32)


def kernel(states, graph_ids, gate_W, gate_b, out_W, out_b):
    w4t = jnp.concatenate(
        [gate_W, out_W, jnp.zeros((D, 1), jnp.float32)], axis=1).T
    b4 = jnp.concatenate(
        [gate_b, out_b, jnp.zeros((1,), jnp.float32)]).reshape(4, 1)

    shp = jax.ShapeDtypeStruct((NBLK, 1, BLK), jnp.float32)
    v0, v1, v2 = pl.pallas_call(
        _stage1_body,
        grid=(NBLK,),
        in_specs=[
            pl.BlockSpec((BLK, D), lambda i: (jnp.minimum(i, LAST_X_BLK), 0)),
            pl.BlockSpec((4, D), lambda i: (0, 0)),
            pl.BlockSpec((4, 1), lambda i: (0, 0)),
        ],
        out_specs=[
            pl.BlockSpec((1, 1, BLK), lambda i: (i, 0, 0)),
            pl.BlockSpec((1, 1, BLK), lambda i: (i, 0, 0)),
            pl.BlockSpec((1, 1, BLK), lambda i: (i, 0, 0)),
        ],
        out_shape=[shp, shp, shp],
    )(states, w4t, b4)

    ids_pad = jnp.concatenate(
        [graph_ids.astype(jnp.int32),
         jnp.full((NPAD - N,), G, jnp.int32)]).reshape(NW, NCH, CS)
    zeros = jnp.zeros((GPS,), jnp.float32)

    partials = _stage2(v0.reshape(NPAD), v1.reshape(NPAD), v2.reshape(NPAD),
                       ids_pad, zeros).reshape(NC, 3, GPS)

    return pl.pallas_call(
        _stage3_body,
        in_specs=[pl.BlockSpec((NC, 3, GPS), lambda: (0, 0, 0))],
        out_specs=pl.BlockSpec((G, 2), lambda: (0, 0)),
        out_shape=jax.ShapeDtypeStruct((G, 2), jnp.float32),
    )(partials)


# speedup vs baseline: 9.9415x; 1.3942x over previous
import functools

import jax
import jax.numpy as jnp
from jax import lax
from jax.experimental import pallas as pl
from jax.experimental.pallas import tpu as pltpu
from jax.experimental.pallas import tpu_sc as plsc

N = 100000
D = 512
G = 512
GPS = 520
BLK = 6400
NBLK = 16
NC, NS = 2, 16
NW = NC * NS
CS = 128
NCH = 25
CPT = NCH * CS
NPAD = NW * CPT
LAST_X_BLK = (N - 1) // BLK


def _stage1_body(x_ref, wt_ref, b_ref, p0_ref, p1_ref, p2_ref):
    x = x_ref[...]
    yt = lax.dot_general(wt_ref[...], x, (((1,), (1,)), ((), ())),
                         preferred_element_type=jnp.float32)
    yt = yt + b_ref[...]
    vals = jnp.exp(yt[0:1])
    p0_ref[0] = vals
    p1_ref[0] = yt[1:2] * vals
    p2_ref[0] = yt[2:3] * vals


def _stage2_body(v0_hbm, v1_hbm, v2_hbm, ids_hbm, zeros_hbm, part_hbm,
                 v0_v, v1_v, v2_v, ids_v, stage_v, acc0, acc1, acc2):
    c = lax.axis_index("c")
    s = lax.axis_index("s")
    w = s * NC + c
    base = w * CPT

    @pl.when(s == 0)
    def _zero():
        pltpu.sync_copy(zeros_hbm, stage_v)
        pltpu.sync_copy(stage_v, acc0)
        pltpu.sync_copy(stage_v, acc1)
        pltpu.sync_copy(stage_v, acc2)

    plsc.subcore_barrier()
    pltpu.sync_copy(v0_hbm.at[pl.ds(base, CPT)], v0_v)
    pltpu.sync_copy(v1_hbm.at[pl.ds(base, CPT)], v1_v)
    pltpu.sync_copy(v2_hbm.at[pl.ds(base, CPT)], v2_v)
    pltpu.sync_copy(ids_hbm.at[w], ids_v)
    for j in range(NCH):
        idsj = ids_v.at[j]
        sl = pl.ds(j * CS, CS)
        pltpu.sync_copy(v0_v.at[sl], acc0.at[idsj], add=True)
        pltpu.sync_copy(v1_v.at[sl], acc1.at[idsj], add=True)
        pltpu.sync_copy(v2_v.at[sl], acc2.at[idsj], add=True)
    plsc.subcore_barrier()

    @pl.when(s == 0)
    def _emit():
        pltpu.sync_copy(acc0, stage_v)
        pltpu.sync_copy(stage_v, part_hbm.at[pl.ds(c * 3 * GPS, GPS)])
        pltpu.sync_copy(acc1, stage_v)
        pltpu.sync_copy(stage_v, part_hbm.at[pl.ds((c * 3 + 1) * GPS, GPS)])
        pltpu.sync_copy(acc2, stage_v)
        pltpu.sync_copy(stage_v, part_hbm.at[pl.ds((c * 3 + 2) * GPS, GPS)])


_stage2 = functools.partial(
    pl.kernel,
    out_type=jax.ShapeDtypeStruct((NC * 3 * GPS,), jnp.float32),
    mesh=plsc.VectorSubcoreMesh(core_axis_name="c", subcore_axis_name="s"),
    scratch_types=[
        pltpu.VMEM((CPT,), jnp.float32),
        pltpu.VMEM((CPT,), jnp.float32),
        pltpu.VMEM((CPT,), jnp.float32),
        pltpu.VMEM((NCH, CS), jnp.int32),
        pltpu.VMEM((GPS,), jnp.float32),
        pltpu.VMEM_SHARED((GPS,), jnp.float32),
        pltpu.VMEM_SHARED((GPS,), jnp.float32),
        pltpu.VMEM_SHARED((GPS,), jnp.float32),
    ],
)(_stage2_body)


def _stage3_body(part_ref, out_ref):
    a = part_ref[0] + part_ref[1]
    gr = a[1:3, 0:G] / (a[0:1, 0:G] + 1e-16)
    e2 = (lax.broadcasted_iota(jnp.int32, (2, 2), 0)
          == lax.broadcasted_iota(jnp.int32, (2, 2), 1)).astype(jnp.float32)
    out_ref[...] = lax.dot_general(gr, e2, (((0,), (0,)), ((), ())),
                                   preferred_element_type=jnp.float32)


def kernel(states, graph_ids, gate_W, gate_b, out_W, out_b):
    w4t = jnp.concatenate(
        [gate_W, out_W, jnp.zeros((D, 1), jnp.float32)], axis=1).T
    b4 = jnp.concatenate(
        [gate_b, out_b, jnp.zeros((1,), jnp.float32)]).reshape(4, 1)

    shp = jax.ShapeDtypeStruct((NBLK, 1, BLK), jnp.float32)
    v0, v1, v2 = pl.pallas_call(
        _stage1_body,
        grid=(NBLK,),
        in_specs=[
            pl.BlockSpec((BLK, D), lambda i: (jnp.minimum(i, LAST_X_BLK), 0)),
            pl.BlockSpec((4, D), lambda i: (0, 0)),
            pl.BlockSpec((4, 1), lambda i: (0, 0)),
        ],
        out_specs=[
            pl.BlockSpec((1, 1, BLK), lambda i: (i, 0, 0)),
            pl.BlockSpec((1, 1, BLK), lambda i: (i, 0, 0)),
            pl.BlockSpec((1, 1, BLK), lambda i: (i, 0, 0)),
        ],
        out_shape=[shp, shp, shp],
    )(states, w4t, b4)

    ids_pad = jnp.concatenate(
        [graph_ids.astype(jnp.int32),
         jnp.full((NPAD - N,), G, jnp.int32)]).reshape(NW, NCH, CS)
    zeros = jnp.zeros((GPS,), jnp.float32)

    partials = _stage2(v0.reshape(NPAD), v1.reshape(NPAD), v2.reshape(NPAD),
                       ids_pad, zeros).reshape(NC, 3, GPS)

    return pl.pallas_call(
        _stage3_body,
        in_specs=[pl.BlockSpec((NC, 3, GPS), lambda: (0, 0, 0))],
        out_specs=pl.BlockSpec((G, 2), lambda: (0, 0)),
        out_shape=jax.ShapeDtypeStruct((G, 2), jnp.float32),
    )(partials)


# speedup vs baseline: 10.0568x; 1.0116x over previous
import functools

import jax
import jax.numpy as jnp
from jax import lax
from jax.experimental import pallas as pl
from jax.experimental.pallas import tpu as pltpu
from jax.experimental.pallas import tpu_sc as plsc

N = 100000
D = 512
G = 512
GPS = 520
BLK = 6400
NBLK = 16
NC, NS = 2, 16
NW = NC * NS
CS = 640
NCH = 5
CPT = NCH * CS
NPAD = NW * CPT
LAST_X_BLK = (N - 1) // BLK


def _stage1_body(x_ref, wt_ref, b_ref, p0_ref, p1_ref, p2_ref):
    x = x_ref[...]
    yt = lax.dot_general(wt_ref[...], x, (((1,), (1,)), ((), ())),
                         preferred_element_type=jnp.float32)
    yt = yt + b_ref[...]
    vals = jnp.exp(yt[0:1])
    p0_ref[0] = vals
    p1_ref[0] = yt[1:2] * vals
    p2_ref[0] = yt[2:3] * vals


def _stage2_body(v0_hbm, v1_hbm, v2_hbm, ids_hbm, zeros_hbm, part_hbm,
                 v0_v, v1_v, v2_v, ids_v, stage_v, acc0, acc1, acc2):
    c = lax.axis_index("c")
    s = lax.axis_index("s")
    w = s * NC + c
    base = w * CPT

    @pl.when(s == 0)
    def _zero():
        pltpu.sync_copy(zeros_hbm, stage_v)
        pltpu.sync_copy(stage_v, acc0)
        pltpu.sync_copy(stage_v, acc1)
        pltpu.sync_copy(stage_v, acc2)

    plsc.subcore_barrier()
    pltpu.sync_copy(v0_hbm.at[pl.ds(base, CPT)], v0_v)
    pltpu.sync_copy(v1_hbm.at[pl.ds(base, CPT)], v1_v)
    pltpu.sync_copy(v2_hbm.at[pl.ds(base, CPT)], v2_v)
    pltpu.sync_copy(ids_hbm.at[w], ids_v)
    for j in range(NCH):
        idsj = ids_v.at[pl.ds(j * CS, CS)]
        sl = pl.ds(j * CS, CS)
        pltpu.sync_copy(v0_v.at[sl], acc0.at[idsj], add=True)
        pltpu.sync_copy(v1_v.at[sl], acc1.at[idsj], add=True)
        pltpu.sync_copy(v2_v.at[sl], acc2.at[idsj], add=True)
    plsc.subcore_barrier()

    @pl.when(s == 0)
    def _emit():
        pltpu.sync_copy(acc0, stage_v)
        pltpu.sync_copy(stage_v, part_hbm.at[pl.ds(c * 3 * GPS, GPS)])
        pltpu.sync_copy(acc1, stage_v)
        pltpu.sync_copy(stage_v, part_hbm.at[pl.ds((c * 3 + 1) * GPS, GPS)])
        pltpu.sync_copy(acc2, stage_v)
        pltpu.sync_copy(stage_v, part_hbm.at[pl.ds((c * 3 + 2) * GPS, GPS)])


_stage2 = functools.partial(
    pl.kernel,
    out_type=jax.ShapeDtypeStruct((NC * 3 * GPS,), jnp.float32),
    mesh=plsc.VectorSubcoreMesh(core_axis_name="c", subcore_axis_name="s"),
    scratch_types=[
        pltpu.VMEM((CPT,), jnp.float32),
        pltpu.VMEM((CPT,), jnp.float32),
        pltpu.VMEM((CPT,), jnp.float32),
        pltpu.VMEM((CPT,), jnp.int32),
        pltpu.VMEM((GPS,), jnp.float32),
        pltpu.VMEM_SHARED((GPS,), jnp.float32),
        pltpu.VMEM_SHARED((GPS,), jnp.float32),
        pltpu.VMEM_SHARED((GPS,), jnp.float32),
    ],
)(_stage2_body)


def _stage3_body(part_ref, out_ref):
    a = part_ref[0] + part_ref[1]
    gr = a[1:3, 0:G] / (a[0:1, 0:G] + 1e-16)
    e2 = (lax.broadcasted_iota(jnp.int32, (2, 2), 0)
          == lax.broadcasted_iota(jnp.int32, (2, 2), 1)).astype(jnp.float32)
    out_ref[...] = lax.dot_general(gr, e2, (((0,), (0,)), ((), ())),
                                   preferred_element_type=jnp.float32)


def kernel(states, graph_ids, gate_W, gate_b, out_W, out_b):
    w4t = jnp.concatenate(
        [gate_W, out_W, jnp.zeros((D, 1), jnp.float32)], axis=1).T
    b4 = jnp.concatenate(
        [gate_b, out_b, jnp.zeros((1,), jnp.float32)]).reshape(4, 1)

    shp = jax.ShapeDtypeStruct((NBLK, 1, BLK), jnp.float32)
    v0, v1, v2 = pl.pallas_call(
        _stage1_body,
        grid=(NBLK,),
        in_specs=[
            pl.BlockSpec((BLK, D), lambda i: (jnp.minimum(i, LAST_X_BLK), 0)),
            pl.BlockSpec((4, D), lambda i: (0, 0)),
            pl.BlockSpec((4, 1), lambda i: (0, 0)),
        ],
        out_specs=[
            pl.BlockSpec((1, 1, BLK), lambda i: (i, 0, 0)),
            pl.BlockSpec((1, 1, BLK), lambda i: (i, 0, 0)),
            pl.BlockSpec((1, 1, BLK), lambda i: (i, 0, 0)),
        ],
        out_shape=[shp, shp, shp],
    )(states, w4t, b4)

    ids_pad = jnp.concatenate(
        [graph_ids.astype(jnp.int32),
         jnp.full((NPAD - N,), G, jnp.int32)]).reshape(NW, CPT)
    zeros = jnp.zeros((GPS,), jnp.float32)

    partials = _stage2(v0.reshape(NPAD), v1.reshape(NPAD), v2.reshape(NPAD),
                       ids_pad, zeros).reshape(NC, 3, GPS)

    return pl.pallas_call(
        _stage3_body,
        in_specs=[pl.BlockSpec((NC, 3, GPS), lambda: (0, 0, 0))],
        out_specs=pl.BlockSpec((G, 2), lambda: (0, 0)),
        out_shape=jax.ShapeDtypeStruct((G, 2), jnp.float32),
    )(partials)


# speedup vs baseline: 10.0633x; 1.0006x over previous
import functools

import jax
import jax.numpy as jnp
from jax import lax
from jax.experimental import pallas as pl
from jax.experimental.pallas import tpu as pltpu
from jax.experimental.pallas import tpu_sc as plsc

N = 100000
D = 512
G = 512
GPS = 520
BLK = 6400
NBLK = 16
NC, NS = 2, 16
NW = NC * NS
CS = 3200
NCH = 1
CPT = NCH * CS
NPAD = NW * CPT
LAST_X_BLK = (N - 1) // BLK


def _stage1_body(x_ref, wt_ref, b_ref, p0_ref, p1_ref, p2_ref):
    x = x_ref[...]
    yt = lax.dot_general(wt_ref[...], x, (((1,), (1,)), ((), ())),
                         preferred_element_type=jnp.float32)
    yt = yt + b_ref[...]
    vals = jnp.exp(yt[0:1])
    p0_ref[0] = vals
    p1_ref[0] = yt[1:2] * vals
    p2_ref[0] = yt[2:3] * vals


def _stage2_body(v0_hbm, v1_hbm, v2_hbm, ids_hbm, zeros_hbm, part_hbm,
                 v0_v, v1_v, v2_v, ids_v, stage_v, acc0, acc1, acc2):
    c = lax.axis_index("c")
    s = lax.axis_index("s")
    w = s * NC + c
    base = w * CPT

    @pl.when(s == 0)
    def _zero():
        pltpu.sync_copy(zeros_hbm, stage_v)
        pltpu.sync_copy(stage_v, acc0)
        pltpu.sync_copy(stage_v, acc1)
        pltpu.sync_copy(stage_v, acc2)

    plsc.subcore_barrier()
    pltpu.sync_copy(v0_hbm.at[pl.ds(base, CPT)], v0_v)
    pltpu.sync_copy(v1_hbm.at[pl.ds(base, CPT)], v1_v)
    pltpu.sync_copy(v2_hbm.at[pl.ds(base, CPT)], v2_v)
    pltpu.sync_copy(ids_hbm.at[w], ids_v)
    for j in range(NCH):
        idsj = ids_v.at[pl.ds(j * CS, CS)]
        sl = pl.ds(j * CS, CS)
        pltpu.sync_copy(v0_v.at[sl], acc0.at[idsj], add=True)
        pltpu.sync_copy(v1_v.at[sl], acc1.at[idsj], add=True)
        pltpu.sync_copy(v2_v.at[sl], acc2.at[idsj], add=True)
    plsc.subcore_barrier()

    @pl.when(s == 0)
    def _emit():
        pltpu.sync_copy(acc0, stage_v)
        pltpu.sync_copy(stage_v, part_hbm.at[pl.ds(c * 3 * GPS, GPS)])
        pltpu.sync_copy(acc1, stage_v)
        pltpu.sync_copy(stage_v, part_hbm.at[pl.ds((c * 3 + 1) * GPS, GPS)])
        pltpu.sync_copy(acc2, stage_v)
        pltpu.sync_copy(stage_v, part_hbm.at[pl.ds((c * 3 + 2) * GPS, GPS)])


_stage2 = functools.partial(
    pl.kernel,
    out_type=jax.ShapeDtypeStruct((NC * 3 * GPS,), jnp.float32),
    mesh=plsc.VectorSubcoreMesh(core_axis_name="c", subcore_axis_name="s"),
    scratch_types=[
        pltpu.VMEM((CPT,), jnp.float32),
        pltpu.VMEM((CPT,), jnp.float32),
        pltpu.VMEM((CPT,), jnp.float32),
        pltpu.VMEM((CPT,), jnp.int32),
        pltpu.VMEM((GPS,), jnp.float32),
        pltpu.VMEM_SHARED((GPS,), jnp.float32),
        pltpu.VMEM_SHARED((GPS,), jnp.float32),
        pltpu.VMEM_SHARED((GPS,), jnp.float32),
    ],
)(_stage2_body)


def _stage3_body(part_ref, out_ref):
    a = part_ref[0] + part_ref[1]
    gr = a[1:3, 0:G] / (a[0:1, 0:G] + 1e-16)
    e2 = (lax.broadcasted_iota(jnp.int32, (2, 2), 0)
          == lax.broadcasted_iota(jnp.int32, (2, 2), 1)).astype(jnp.float32)
    out_ref[...] = lax.dot_general(gr, e2, (((0,), (0,)), ((), ())),
                                   preferred_element_type=jnp.float32)


def kernel(states, graph_ids, gate_W, gate_b, out_W, out_b):
    w4t = jnp.concatenate(
        [gate_W, out_W, jnp.zeros((D, 1), jnp.float32)], axis=1).T
    b4 = jnp.concatenate(
        [gate_b, out_b, jnp.zeros((1,), jnp.float32)]).reshape(4, 1)

    shp = jax.ShapeDtypeStruct((NBLK, 1, BLK), jnp.float32)
    v0, v1, v2 = pl.pallas_call(
        _stage1_body,
        grid=(NBLK,),
        in_specs=[
            pl.BlockSpec((BLK, D), lambda i: (jnp.minimum(i, LAST_X_BLK), 0)),
            pl.BlockSpec((4, D), lambda i: (0, 0)),
            pl.BlockSpec((4, 1), lambda i: (0, 0)),
        ],
        out_specs=[
            pl.BlockSpec((1, 1, BLK), lambda i: (i, 0, 0)),
            pl.BlockSpec((1, 1, BLK), lambda i: (i, 0, 0)),
            pl.BlockSpec((1, 1, BLK), lambda i: (i, 0, 0)),
        ],
        out_shape=[shp, shp, shp],
    )(states, w4t, b4)

    ids_pad = jnp.concatenate(
        [graph_ids.astype(jnp.int32),
         jnp.full((NPAD - N,), G, jnp.int32)]).reshape(NW, CPT)
    zeros = jnp.zeros((GPS,), jnp.float32)

    partials = _stage2(v0.reshape(NPAD), v1.reshape(NPAD), v2.reshape(NPAD),
                       ids_pad, zeros).reshape(NC, 3, GPS)

    return pl.pallas_call(
        _stage3_body,
        in_specs=[pl.BlockSpec((NC, 3, GPS), lambda: (0, 0, 0))],
        out_specs=pl.BlockSpec((G, 2), lambda: (0, 0)),
        out_shape=jax.ShapeDtypeStruct((G, 2), jnp.float32),
    )(partials)


# speedup vs baseline: 10.1487x; 1.0085x over previous
import functools

import jax
import jax.numpy as jnp
from jax import lax
from jax.experimental import pallas as pl
from jax.experimental.pallas import tpu as pltpu
from jax.experimental.pallas import tpu_sc as plsc

N = 100000
D = 512
G = 512
GPS = 520
BLK = 6400
NBLK = 16
NC, NS = 2, 16
NW = NC * NS
CS = 3200
NCH = 1
CPT = NCH * CS
NPAD = NW * CPT
LAST_X_BLK = (N - 1) // BLK
LW = NW - 1
NVALID = N - LW * CPT


def _stage1_body(x_ref, wt_ref, b_ref, p0_ref, p1_ref, p2_ref):
    x = x_ref[...]
    yt = lax.dot_general(wt_ref[...], x, (((1,), (1,)), ((), ())),
                         preferred_element_type=jnp.float32)
    yt = yt + b_ref[...]
    vals = jnp.exp(yt[0:1])
    p0_ref[0] = vals
    p1_ref[0] = yt[1:2] * vals
    p2_ref[0] = yt[2:3] * vals


def _stage2_body(v0_hbm, v1_hbm, v2_hbm, ids_hbm, dump_hbm, zeros_hbm,
                 part_hbm, v0_v, v1_v, v2_v, ids_v, stage_v,
                 acc0, acc1, acc2):
    c = lax.axis_index("c")
    s = lax.axis_index("s")
    w = s * NC + c
    base = w * CPT
    accs = [acc0, acc1, acc2]

    @pl.when(s == 0)
    def _zero():
        pltpu.sync_copy(zeros_hbm, stage_v)
        pltpu.sync_copy(stage_v, acc0)
        pltpu.sync_copy(stage_v, acc1)
        pltpu.sync_copy(stage_v, acc2)
    plsc.subcore_barrier()

    pltpu.sync_copy(v0_hbm.at[pl.ds(base, CPT)], v0_v)
    pltpu.sync_copy(v1_hbm.at[pl.ds(base, CPT)], v1_v)
    pltpu.sync_copy(v2_hbm.at[pl.ds(base, CPT)], v2_v)
    @pl.when(w < NW - 1)
    def _ids_full():
        pltpu.sync_copy(ids_hbm.at[pl.ds(base, CPT)], ids_v)

    @pl.when(w == NW - 1)
    def _ids_tail():
        pltpu.sync_copy(ids_hbm.at[pl.ds(LW * CPT, NVALID)],
                        ids_v.at[pl.ds(0, NVALID)])
        pltpu.sync_copy(dump_hbm, ids_v.at[pl.ds(NVALID, NPAD - N)])

    for j in range(NCH):
        idsj = ids_v.at[pl.ds(j * CS, CS)]
        sl = pl.ds(j * CS, CS)
        pltpu.sync_copy(v0_v.at[sl], acc0.at[idsj], add=True)
        pltpu.sync_copy(v1_v.at[sl], acc1.at[idsj], add=True)
        pltpu.sync_copy(v2_v.at[sl], acc2.at[idsj], add=True)
    plsc.subcore_barrier()

    @pl.when(s == 0)
    def _emit():
        for k in range(3):
            pltpu.sync_copy(accs[k], stage_v)
            pltpu.sync_copy(stage_v,
                            part_hbm.at[pl.ds((c * 3 + k) * GPS, GPS)])


_stage2 = functools.partial(
    pl.kernel,
    out_type=jax.ShapeDtypeStruct((NC * 3 * GPS,), jnp.float32),
    mesh=plsc.VectorSubcoreMesh(core_axis_name="c", subcore_axis_name="s"),
    scratch_types=[
        pltpu.VMEM((CPT,), jnp.float32),
        pltpu.VMEM((CPT,), jnp.float32),
        pltpu.VMEM((CPT,), jnp.float32),
        pltpu.VMEM((CPT,), jnp.int32),
        pltpu.VMEM((GPS,), jnp.float32),
        pltpu.VMEM_SHARED((GPS,), jnp.float32),
        pltpu.VMEM_SHARED((GPS,), jnp.float32),
        pltpu.VMEM_SHARED((GPS,), jnp.float32),
    ],
)(_stage2_body)


def _stage3_body(part_ref, out_ref):
    a = part_ref[0] + part_ref[1]
    gr = a[1:3, 0:G] / (a[0:1, 0:G] + 1e-16)
    e2 = (lax.broadcasted_iota(jnp.int32, (2, 2), 0)
          == lax.broadcasted_iota(jnp.int32, (2, 2), 1)).astype(jnp.float32)
    out_ref[...] = lax.dot_general(gr, e2, (((0,), (0,)), ((), ())),
                                   preferred_element_type=jnp.float32)


def kernel(states, graph_ids, gate_W, gate_b, out_W, out_b):
    w4t = jnp.concatenate(
        [gate_W, out_W, jnp.zeros((D, 1), jnp.float32)], axis=1).T
    b4 = jnp.concatenate(
        [gate_b, out_b, jnp.zeros((1,), jnp.float32)]).reshape(4, 1)

    shp = jax.ShapeDtypeStruct((NBLK, 1, BLK), jnp.float32)
    v0, v1, v2 = pl.pallas_call(
        _stage1_body,
        grid=(NBLK,),
        in_specs=[
            pl.BlockSpec((BLK, D), lambda i: (jnp.minimum(i, LAST_X_BLK), 0)),
            pl.BlockSpec((4, D), lambda i: (0, 0)),
            pl.BlockSpec((4, 1), lambda i: (0, 0)),
        ],
        out_specs=[
            pl.BlockSpec((1, 1, BLK), lambda i: (i, 0, 0)),
            pl.BlockSpec((1, 1, BLK), lambda i: (i, 0, 0)),
            pl.BlockSpec((1, 1, BLK), lambda i: (i, 0, 0)),
        ],
        out_shape=[shp, shp, shp],
    )(states, w4t, b4)

    dump = jnp.full((NPAD - N,), G, jnp.int32)
    zeros = jnp.zeros((GPS,), jnp.float32)

    partials = _stage2(v0.reshape(NPAD), v1.reshape(NPAD), v2.reshape(NPAD),
                       graph_ids.astype(jnp.int32), dump,
                       zeros).reshape(NC, 3, GPS)

    return pl.pallas_call(
        _stage3_body,
        in_specs=[pl.BlockSpec((NC, 3, GPS), lambda: (0, 0, 0))],
        out_specs=pl.BlockSpec((G, 2), lambda: (0, 0)),
        out_shape=jax.ShapeDtypeStruct((G, 2), jnp.float32),
    )(partials)
